# baseline (speedup 1.0000x reference)
import functools

import jax
import jax.numpy as jnp
from jax import lax
from jax.experimental import pallas as pl
from jax.experimental.pallas import tpu as pltpu
from jax.experimental.pallas import tpu_sc as plsc


def _make_sc_pool(B, CTXP, CTX, D, NC=2, NS=16):
    NW = NC * NS
    b_per_w = B // NW
    mesh = plsc.VectorSubcoreMesh(core_axis_name="c", subcore_axis_name="s")

    @functools.partial(
        pl.kernel,
        out_type=jax.ShapeDtypeStruct((B, D), jnp.float32),
        mesh=mesh,
        scratch_types=[
            pltpu.VMEM((b_per_w, CTXP), jnp.int32),
            pltpu.VMEM((2 * CTXP, D), jnp.float32),
            pltpu.VMEM((b_per_w, D), jnp.float32),
            pltpu.SemaphoreType.DMA,
            pltpu.SemaphoreType.DMA,
        ],
        compiler_params=pltpu.CompilerParams(use_tc_tiling_on_sc=False),
    )
    def sc_pool(idx_hbm, emb_hbm, out_hbm, idx_v, rows_v, acc_v, sem0, sem1):
        wid = lax.axis_index("s") * NC + lax.axis_index("c")
        base = wid * b_per_w
        pltpu.sync_copy(idx_hbm.at[pl.ds(base, b_per_w)], idx_v)
        sems = (sem0, sem1)

        def start(bi, slot):
            return pltpu.async_copy(
                emb_hbm.at[idx_v.at[bi]],
                rows_v.at[pl.ds(slot * CTXP, CTXP)],
                sems[slot],
            )

        handles = {0: start(0, 0)}
        for bi in range(b_per_w):
            slot = bi & 1
            handles.pop(bi).wait()
            if bi + 1 < b_per_w:
                handles[bi + 1] = start(bi + 1, slot ^ 1)
            rbase = slot * CTXP

            def body(r, accs, rbase=rbase):
                return tuple(
                    a + rows_v[rbase + r, pl.ds(16 * c, 16)]
                    for c, a in enumerate(accs)
                )

            z = jnp.zeros((16,), jnp.float32)
            accs = lax.fori_loop(0, CTX, body, (z,) * (D // 16))
            for c, a in enumerate(accs):
                acc_v[bi, pl.ds(16 * c, 16)] = a
        pltpu.sync_copy(acc_v, out_hbm.at[pl.ds(base, b_per_w)])

    return sc_pool


def _mm_kernel(scale, s_ref, w_ref, b_ref, o_ref):
    o_ref[...] = (
        lax.dot_general(
            s_ref[...] * scale,
            w_ref[...],
            dimension_numbers=(((1,), (1,)), ((), ())),
            preferred_element_type=jnp.float32,
        )
        + b_ref[...]
    )


def _project(sums, W, b2d, ctx, tile_v):
    B, D = sums.shape
    V = W.shape[0]
    nv = pl.cdiv(V, tile_v)
    return pl.pallas_call(
        functools.partial(_mm_kernel, 1.0 / ctx),
        grid=(nv,),
        in_specs=[
            pl.BlockSpec((B, D), lambda j: (0, 0)),
            pl.BlockSpec((tile_v, D), lambda j: (j, 0)),
            pl.BlockSpec((1, tile_v), lambda j: (0, j)),
        ],
        out_specs=pl.BlockSpec((B, tile_v), lambda j: (0, j)),
        out_shape=jax.ShapeDtypeStruct((B, V), jnp.float32),
        compiler_params=pltpu.CompilerParams(
            dimension_semantics=("arbitrary",),
        ),
    )(sums, W, b2d)


def kernel(context_idxs, emb_table, W, b):
    B, CTX = context_idxs.shape
    V, D = emb_table.shape
    idx = context_idxs.astype(jnp.int32)
    CTXP = (CTX + 7) // 8 * 8
    if CTXP != CTX:
        idx = jnp.pad(idx, ((0, 0), (0, CTXP - CTX)))
    sums = _make_sc_pool(B, CTXP, CTX, D)(idx, emb_table)
    return _project(sums, W, b.reshape(1, V), CTX, 2048)


# speedup vs baseline: 1.1893x; 1.1893x over previous
import functools

import jax
import jax.numpy as jnp
from jax import lax
from jax.experimental import pallas as pl
from jax.experimental.pallas import tpu as pltpu
from jax.experimental.pallas import tpu_sc as plsc


def _make_sc_pool(B, CTX, D, EPC=2, NC=2, NS=16):
    NW = NC * NS
    b_per_w = B // NW
    n_chunk = b_per_w // EPC
    rows_per_chunk = EPC * CTX
    mesh = plsc.VectorSubcoreMesh(core_axis_name="c", subcore_axis_name="s")

    @functools.partial(
        pl.kernel,
        out_type=jax.ShapeDtypeStruct((B, D), jnp.float32),
        mesh=mesh,
        scratch_types=[
            pltpu.VMEM((n_chunk, rows_per_chunk), jnp.int32),
            pltpu.VMEM((b_per_w * CTX, D), jnp.float32),
            pltpu.VMEM((b_per_w, D), jnp.float32),
            pltpu.SemaphoreType.DMA,
        ],
        compiler_params=pltpu.CompilerParams(use_tc_tiling_on_sc=False),
    )
    def sc_pool(idx_hbm, emb_hbm, out_hbm, idx_v, rows_v, acc_v, sem):
        wid = lax.axis_index("s") * NC + lax.axis_index("c")
        base = wid * n_chunk
        pltpu.sync_copy(idx_hbm.at[pl.ds(base, n_chunk)], idx_v)
        handles = [
            pltpu.async_copy(
                emb_hbm.at[idx_v.at[k]],
                rows_v.at[pl.ds(k * rows_per_chunk, rows_per_chunk)],
                sem,
            )
            for k in range(n_chunk)
        ]
        for h in handles:
            h.wait()
        for bi in range(b_per_w):
            rbase = bi * CTX

            def body(r, accs, rbase=rbase):
                return tuple(
                    a + rows_v[rbase + r, pl.ds(16 * c, 16)]
                    for c, a in enumerate(accs)
                )

            z = jnp.zeros((16,), jnp.float32)
            accs = lax.fori_loop(0, CTX, body, (z,) * (D // 16))
            for c, a in enumerate(accs):
                acc_v[bi, pl.ds(16 * c, 16)] = a
        pltpu.sync_copy(acc_v, out_hbm.at[pl.ds(wid * b_per_w, b_per_w)])

    return sc_pool


def _mm_kernel(scale, s_ref, w_ref, b_ref, o_ref):
    o_ref[...] = (
        lax.dot_general(
            s_ref[...] * scale,
            w_ref[...],
            dimension_numbers=(((1,), (1,)), ((), ())),
            preferred_element_type=jnp.float32,
        )
        + b_ref[...]
    )


def _project(sums, W, b2d, ctx, tile_v):
    B, D = sums.shape
    V = W.shape[0]
    nv = pl.cdiv(V, tile_v)
    return pl.pallas_call(
        functools.partial(_mm_kernel, 1.0 / ctx),
        grid=(nv,),
        in_specs=[
            pl.BlockSpec((B, D), lambda j: (0, 0)),
            pl.BlockSpec((tile_v, D), lambda j: (j, 0)),
            pl.BlockSpec((1, tile_v), lambda j: (0, j)),
        ],
        out_specs=pl.BlockSpec((B, tile_v), lambda j: (0, j)),
        out_shape=jax.ShapeDtypeStruct((B, V), jnp.float32),
        compiler_params=pltpu.CompilerParams(
            dimension_semantics=("arbitrary",),
        ),
    )(sums, W, b2d)


def kernel(context_idxs, emb_table, W, b):
    B, CTX = context_idxs.shape
    V, D = emb_table.shape
    EPC = 2
    idx = context_idxs.astype(jnp.int32).reshape(B // EPC, EPC * CTX)
    sums = _make_sc_pool(B, CTX, D, EPC)(idx, emb_table)
    return _project(sums, W, b.reshape(1, V), CTX, 2048)


# speedup vs baseline: 1.1911x; 1.0015x over previous
import functools

import jax
import jax.numpy as jnp
from jax import lax
from jax.experimental import pallas as pl
from jax.experimental.pallas import tpu as pltpu
from jax.experimental.pallas import tpu_sc as plsc


def _make_sc_pool(B, CTX, D, EPC=2, NC=2, NS=16):
    NW = NC * NS
    b_per_w = B // NW
    n_chunk = b_per_w // EPC
    rows_per_chunk = EPC * CTX
    mesh = plsc.VectorSubcoreMesh(core_axis_name="c", subcore_axis_name="s")

    @functools.partial(
        pl.kernel,
        out_type=jax.ShapeDtypeStruct((B, D), jnp.float32),
        mesh=mesh,
        scratch_types=[
            pltpu.VMEM((n_chunk, rows_per_chunk), jnp.int32),
            pltpu.VMEM((b_per_w * CTX, D), jnp.float32),
            pltpu.VMEM((b_per_w, D), jnp.float32),
            pltpu.SemaphoreType.DMA,
        ],
        compiler_params=pltpu.CompilerParams(use_tc_tiling_on_sc=False),
    )
    def sc_pool(idx_hbm, emb_hbm, out_hbm, idx_v, rows_v, acc_v, sem):
        wid = lax.axis_index("s") * NC + lax.axis_index("c")
        base = wid * n_chunk
        pltpu.sync_copy(idx_hbm.at[pl.ds(base, n_chunk)], idx_v)
        handles = [
            pltpu.async_copy(
                emb_hbm.at[idx_v.at[k]],
                rows_v.at[pl.ds(k * rows_per_chunk, rows_per_chunk)],
                sem,
            )
            for k in range(n_chunk)
        ]
        for h in handles:
            h.wait()
        for bi in range(b_per_w):
            rbase = bi * CTX

            def body(r, accs, rbase=rbase):
                return tuple(
                    a + rows_v[rbase + r, pl.ds(16 * c, 16)]
                    for c, a in enumerate(accs)
                )

            z = jnp.zeros((16,), jnp.float32)
            accs = lax.fori_loop(0, CTX, body, (z,) * (D // 16))
            for c, a in enumerate(accs):
                acc_v[bi, pl.ds(16 * c, 16)] = a
        pltpu.sync_copy(acc_v, out_hbm.at[pl.ds(wid * b_per_w, b_per_w)])

    return sc_pool


def _mm_kernel(scale, s_ref, w_ref, b_ref, o_ref):
    o_ref[...] = (
        lax.dot_general(
            s_ref[...] * scale,
            w_ref[...],
            dimension_numbers=(((1,), (1,)), ((), ())),
            preferred_element_type=jnp.float32,
        )
        + b_ref[...]
    )


def _project(sums, W, b2d, ctx, tile_v):
    B, D = sums.shape
    V = W.shape[0]
    nv = pl.cdiv(V, tile_v)
    return pl.pallas_call(
        functools.partial(_mm_kernel, 1.0 / ctx),
        grid=(nv,),
        in_specs=[
            pl.BlockSpec((B, D), lambda j: (0, 0)),
            pl.BlockSpec((tile_v, D), lambda j: (j, 0)),
            pl.BlockSpec((1, tile_v), lambda j: (0, j)),
        ],
        out_specs=pl.BlockSpec((B, tile_v), lambda j: (0, j)),
        out_shape=jax.ShapeDtypeStruct((B, V), jnp.float32),
        compiler_params=pltpu.CompilerParams(
            dimension_semantics=("parallel",),
        ),
    )(sums, W, b2d)


def kernel(context_idxs, emb_table, W, b):
    B, CTX = context_idxs.shape
    V, D = emb_table.shape
    EPC = 2
    idx = context_idxs.astype(jnp.int32).reshape(B // EPC, EPC * CTX)
    sums = _make_sc_pool(B, CTX, D, EPC)(idx, emb_table)
    return _project(sums, W, b.reshape(1, V), CTX, 2048)


# speedup vs baseline: 3.2010x; 2.6874x over previous
import functools

import jax
import jax.numpy as jnp
from jax import lax
from jax.experimental import pallas as pl
from jax.experimental.pallas import tpu as pltpu
from jax.experimental.pallas import tpu_sc as plsc


def _make_sc_pool(B, CTX, D, EPC=2, NC=2, NS=16):
    NW = NC * NS
    b_per_w = B // NW
    n_chunk = b_per_w // EPC
    rows_per_chunk = EPC * CTX
    mesh = plsc.VectorSubcoreMesh(core_axis_name="c", subcore_axis_name="s")

    @functools.partial(
        pl.kernel,
        out_type=jax.ShapeDtypeStruct((B, D), jnp.float32),
        mesh=mesh,
        scratch_types=[
            pltpu.VMEM((n_chunk, rows_per_chunk), jnp.int32),
            pltpu.VMEM((b_per_w * CTX, D), jnp.float32),
            pltpu.VMEM((b_per_w, D), jnp.float32),
            pltpu.SemaphoreType.DMA,
        ],
        compiler_params=pltpu.CompilerParams(use_tc_tiling_on_sc=False),
    )
    def sc_pool(idx_hbm, emb_hbm, out_hbm, idx_v, rows_v, acc_v, sem):
        wid = lax.axis_index("s") * NC + lax.axis_index("c")
        base = wid * n_chunk
        pltpu.sync_copy(idx_hbm.at[pl.ds(base, n_chunk)], idx_v)
        handles = [
            pltpu.async_copy(
                emb_hbm.at[idx_v.at[k]],
                rows_v.at[pl.ds(k * rows_per_chunk, rows_per_chunk)],
                sem,
            )
            for k in range(n_chunk)
        ]
        for h in handles:
            h.wait()
        for bi in range(b_per_w):
            rbase = bi * CTX

            def body(r, accs, rbase=rbase):
                return tuple(
                    a + rows_v[rbase + r, pl.ds(16 * c, 16)]
                    for c, a in enumerate(accs)
                )

            z = jnp.zeros((16,), jnp.float32)
            accs = lax.fori_loop(0, CTX, body, (z,) * (D // 16))
            for c, a in enumerate(accs):
                acc_v[bi, pl.ds(16 * c, 16)] = a
        pltpu.sync_copy(acc_v, out_hbm.at[pl.ds(wid * b_per_w, b_per_w)])

    return sc_pool


def _mm_kernel(s_ref, w_ref, b_ref, o_ref):
    lhs = jnp.concatenate([w_ref[...], b_ref[...]], axis=0)
    o_ref[...] = lax.dot_general(
        lhs,
        s_ref[...],
        dimension_numbers=(((0,), (1,)), ((), ())),
        preferred_element_type=jnp.float32,
    )


def _project(sums, W_t, b2d, ctx, tile_v):
    D, V = W_t.shape
    B = sums.shape[0]
    nv = pl.cdiv(V, tile_v)
    rhs = jnp.concatenate(
        [sums * (1.0 / ctx), jnp.ones((B, 1), jnp.float32)], axis=1
    )
    out_t = pl.pallas_call(
        _mm_kernel,
        grid=(nv,),
        in_specs=[
            pl.BlockSpec((B, D + 1), lambda j: (0, 0)),
            pl.BlockSpec((D, tile_v), lambda j: (0, j)),
            pl.BlockSpec((1, tile_v), lambda j: (0, j)),
        ],
        out_specs=pl.BlockSpec((tile_v, B), lambda j: (j, 0)),
        out_shape=jax.ShapeDtypeStruct((V, B), jnp.float32),
        compiler_params=pltpu.CompilerParams(
            dimension_semantics=("parallel",),
        ),
    )(rhs, W_t, b2d)
    return jnp.transpose(out_t)


def kernel(context_idxs, emb_table, W, b):
    B, CTX = context_idxs.shape
    V, D = emb_table.shape
    EPC = 2
    idx = context_idxs.astype(jnp.int32).reshape(B // EPC, EPC * CTX)
    sums = _make_sc_pool(B, CTX, D, EPC)(idx, emb_table)
    return _project(sums, W.T, b.reshape(1, V), CTX, 2048)


# speedup vs baseline: 3.3598x; 1.0496x over previous
import functools

import jax
import jax.numpy as jnp
from jax import lax
from jax.experimental import pallas as pl
from jax.experimental.pallas import tpu as pltpu
from jax.experimental.pallas import tpu_sc as plsc

_LANES = 128


def _prep_kernel(et_ref, o_ref):
    blk = et_ref[...]
    o_ref[...] = jnp.concatenate(
        [blk.T, jnp.zeros((blk.shape[1], _LANES - blk.shape[0]), jnp.float32)],
        axis=1,
    )


def _widen_table(emb_t, tile_v):
    D, V = emb_t.shape
    return pl.pallas_call(
        _prep_kernel,
        grid=(pl.cdiv(V, tile_v),),
        in_specs=[pl.BlockSpec((D, tile_v), lambda j: (0, j))],
        out_specs=pl.BlockSpec((tile_v, _LANES), lambda j: (j, 0)),
        out_shape=jax.ShapeDtypeStruct((V, _LANES), jnp.float32),
        compiler_params=pltpu.CompilerParams(
            dimension_semantics=("parallel",),
        ),
    )(emb_t)


def _make_sc_pool(B, CTX, D, NC=2, NS=16, PASS_E=8):
    NW = NC * NS
    b_per_w = B // NW
    passes = b_per_w // PASS_E
    rows_per_pass = PASS_E * CTX
    mesh = plsc.VectorSubcoreMesh(core_axis_name="c", subcore_axis_name="s")

    @functools.partial(
        pl.kernel,
        out_type=jax.ShapeDtypeStruct((B, D), jnp.float32),
        mesh=mesh,
        scratch_types=[
            pltpu.VMEM((b_per_w * CTX,), jnp.int32),
            pltpu.VMEM((2 * rows_per_pass, _LANES), jnp.float32),
            pltpu.VMEM((b_per_w, D), jnp.float32),
            pltpu.SemaphoreType.DMA,
        ],
    )
    def sc_pool(idx_hbm, emb_hbm, out_hbm, idx_v, rows_v, acc_v, sem):
        wid = lax.axis_index("s") * NC + lax.axis_index("c")
        pltpu.sync_copy(
            idx_hbm.at[pl.ds(wid * b_per_w * CTX, b_per_w * CTX)], idx_v
        )

        def fire(p, slot):
            handles = []
            pos = 0
            while pos < rows_per_pass:
                n = min(_LANES, rows_per_pass - pos)
                handles.append(
                    pltpu.async_copy(
                        emb_hbm.at[idx_v.at[pl.ds(p * rows_per_pass + pos, n)]],
                        rows_v.at[pl.ds(slot * rows_per_pass + pos, n)],
                        sem,
                    )
                )
                pos += n
            return handles

        pending = fire(0, 0)
        for p in range(passes):
            slot = p & 1
            for h in pending:
                h.wait()
            if p + 1 < passes:
                pending = fire(p + 1, slot ^ 1)
            for e in range(PASS_E):
                bi = p * PASS_E + e
                rbase = slot * rows_per_pass + e * CTX

                def body(r, accs, rbase=rbase):
                    return tuple(
                        a + rows_v[rbase + r, pl.ds(16 * c, 16)]
                        for c, a in enumerate(accs)
                    )

                z = jnp.zeros((16,), jnp.float32)
                accs = lax.fori_loop(0, CTX, body, (z,) * (D // 16))
                for c, a in enumerate(accs):
                    acc_v[bi, pl.ds(16 * c, 16)] = a
        pltpu.sync_copy(acc_v, out_hbm.at[pl.ds(wid * b_per_w, b_per_w)])

    return sc_pool


def _mm_kernel(s_ref, w_ref, b_ref, o_ref):
    lhs = jnp.concatenate([w_ref[...], b_ref[...]], axis=0)
    o_ref[...] = lax.dot_general(
        lhs,
        s_ref[...],
        dimension_numbers=(((0,), (1,)), ((), ())),
        preferred_element_type=jnp.float32,
    )


def _project(sums, W_t, b2d, ctx, tile_v):
    D, V = W_t.shape
    B = sums.shape[0]
    rhs = jnp.concatenate(
        [sums * (1.0 / ctx), jnp.ones((B, 1), jnp.float32)], axis=1
    )
    out_t = pl.pallas_call(
        _mm_kernel,
        grid=(pl.cdiv(V, tile_v),),
        in_specs=[
            pl.BlockSpec((B, D + 1), lambda j: (0, 0)),
            pl.BlockSpec((D, tile_v), lambda j: (0, j)),
            pl.BlockSpec((1, tile_v), lambda j: (0, j)),
        ],
        out_specs=pl.BlockSpec((tile_v, B), lambda j: (j, 0)),
        out_shape=jax.ShapeDtypeStruct((V, B), jnp.float32),
        compiler_params=pltpu.CompilerParams(
            dimension_semantics=("parallel",),
        ),
    )(rhs, W_t, b2d)
    return jnp.transpose(out_t)


def kernel(context_idxs, emb_table, W, b):
    B, CTX = context_idxs.shape
    V, D = emb_table.shape
    idx_lin = context_idxs.astype(jnp.int32).reshape(B * CTX)
    emb128 = _widen_table(emb_table.T, 2048)
    sums = _make_sc_pool(B, CTX, D)(idx_lin, emb128)
    return _project(sums, W.T, b.reshape(1, V), CTX, 2048)


# speedup vs baseline: 3.6865x; 1.0972x over previous
import functools

import jax
import jax.numpy as jnp
from jax import lax
from jax.experimental import pallas as pl
from jax.experimental.pallas import tpu as pltpu
from jax.experimental.pallas import tpu_sc as plsc

_LANES = 128


def _prep_kernel(et_ref, o_ref):
    blk = et_ref[...]
    o_ref[...] = jnp.concatenate(
        [blk.T, jnp.zeros((blk.shape[1], _LANES - blk.shape[0]), jnp.float32)],
        axis=1,
    )


def _widen_table(emb_t, tile_v):
    D, V = emb_t.shape
    return pl.pallas_call(
        _prep_kernel,
        grid=(pl.cdiv(V, tile_v),),
        in_specs=[pl.BlockSpec((D, tile_v), lambda j: (0, j))],
        out_specs=pl.BlockSpec((tile_v, _LANES), lambda j: (j, 0)),
        out_shape=jax.ShapeDtypeStruct((V, _LANES), jnp.float32),
        compiler_params=pltpu.CompilerParams(
            dimension_semantics=("parallel",),
        ),
    )(emb_t)


def _make_sc_pool(B, CTX, D, NC=2, NS=16, PASS_E=8):
    NW = NC * NS
    b_per_w = B // NW
    passes = b_per_w // PASS_E
    rows_per_pass = PASS_E * CTX
    mesh = plsc.VectorSubcoreMesh(core_axis_name="c", subcore_axis_name="s")

    @functools.partial(
        pl.kernel,
        out_type=jax.ShapeDtypeStruct((B, D), jnp.float32),
        mesh=mesh,
        scratch_types=[
            pltpu.VMEM((b_per_w * CTX,), jnp.int32),
            pltpu.VMEM((2 * rows_per_pass, _LANES), jnp.float32),
            pltpu.VMEM((b_per_w, D), jnp.float32),
            pltpu.SemaphoreType.DMA,
        ],
    )
    def sc_pool(idx_hbm, emb_hbm, out_hbm, idx_v, rows_v, acc_v, sem):
        wid = lax.axis_index("s") * NC + lax.axis_index("c")
        pltpu.sync_copy(
            idx_hbm.at[pl.ds(wid * b_per_w * CTX, b_per_w * CTX)], idx_v
        )

        def fire(p, slot):
            handles = []
            pos = 0
            while pos < rows_per_pass:
                n = min(_LANES, rows_per_pass - pos)
                handles.append(
                    pltpu.async_copy(
                        emb_hbm.at[idx_v.at[pl.ds(p * rows_per_pass + pos, n)]],
                        rows_v.at[pl.ds(slot * rows_per_pass + pos, n)],
                        sem,
                    )
                )
                pos += n
            return handles

        pending = fire(0, 0)
        for p in range(passes):
            slot = p & 1
            for h in pending:
                h.wait()
            if p + 1 < passes:
                pending = fire(p + 1, slot ^ 1)
            for e in range(PASS_E):
                bi = p * PASS_E + e
                rbase = slot * rows_per_pass + e * CTX

                def body(r, accs, rbase=rbase):
                    return tuple(
                        a + rows_v[rbase + r, pl.ds(16 * c, 16)]
                        for c, a in enumerate(accs)
                    )

                z = jnp.zeros((16,), jnp.float32)
                accs = lax.fori_loop(0, CTX, body, (z,) * (D // 16))
                for c, a in enumerate(accs):
                    acc_v[bi, pl.ds(16 * c, 16)] = a
        pltpu.sync_copy(acc_v, out_hbm.at[pl.ds(wid * b_per_w, b_per_w)])

    return sc_pool


def _mm_kernel(s_ref, w_ref, b_ref, o_ref):
    lhs = jnp.concatenate([w_ref[...], b_ref[...]], axis=0)
    o_ref[...] = lax.dot_general(
        lhs,
        s_ref[...],
        dimension_numbers=(((0,), (1,)), ((), ())),
        preferred_element_type=jnp.float32,
    )


def _project(sums, W_t, b2d, ctx, tile_v):
    D, V = W_t.shape
    B = sums.shape[0]
    rhs = jnp.concatenate(
        [sums * (1.0 / ctx), jnp.ones((B, 1), jnp.float32)], axis=1
    )
    out_t = pl.pallas_call(
        _mm_kernel,
        grid=(pl.cdiv(V, tile_v),),
        in_specs=[
            pl.BlockSpec((B, D + 1), lambda j: (0, 0)),
            pl.BlockSpec((D, tile_v), lambda j: (0, j)),
            pl.BlockSpec((1, tile_v), lambda j: (0, j)),
        ],
        out_specs=pl.BlockSpec((tile_v, B), lambda j: (j, 0)),
        out_shape=jax.ShapeDtypeStruct((V, B), jnp.float32),
        compiler_params=pltpu.CompilerParams(
            dimension_semantics=("parallel",),
        ),
    )(rhs, W_t, b2d)
    return jnp.transpose(out_t)


def kernel(context_idxs, emb_table, W, b):
    B, CTX = context_idxs.shape
    V, D = emb_table.shape
    idx_lin = context_idxs.astype(jnp.int32).reshape(B * CTX)
    emb128 = _widen_table(emb_table.T, 8192)
    sums = _make_sc_pool(B, CTX, D)(idx_lin, emb128)
    return _project(sums, W.T, b.reshape(1, V), CTX, 2048)


# speedup vs baseline: 3.7184x; 1.0087x over previous
import functools

import jax
import jax.numpy as jnp
from jax import lax
from jax.experimental import pallas as pl
from jax.experimental.pallas import tpu as pltpu
from jax.experimental.pallas import tpu_sc as plsc

_LANES = 128


def _prep_kernel(et_ref, o_ref):
    blk = et_ref[...]
    o_ref[...] = jnp.concatenate(
        [blk.T, jnp.zeros((blk.shape[1], _LANES - blk.shape[0]), jnp.float32)],
        axis=1,
    )


def _widen_table(emb_t, tile_v):
    D, V = emb_t.shape
    return pl.pallas_call(
        _prep_kernel,
        grid=(pl.cdiv(V, tile_v),),
        in_specs=[pl.BlockSpec((D, tile_v), lambda j: (0, j))],
        out_specs=pl.BlockSpec((tile_v, _LANES), lambda j: (j, 0)),
        out_shape=jax.ShapeDtypeStruct((V, _LANES), jnp.float32),
        compiler_params=pltpu.CompilerParams(
            dimension_semantics=("parallel",),
        ),
    )(emb_t)


def _make_sc_pool(B, CTX, D, NC=2, NS=16, PASS_E=8):
    NW = NC * NS
    b_per_w = B // NW
    passes = b_per_w // PASS_E
    rows_per_pass = PASS_E * CTX
    mesh = plsc.VectorSubcoreMesh(core_axis_name="c", subcore_axis_name="s")

    @functools.partial(
        pl.kernel,
        out_type=jax.ShapeDtypeStruct((B, D), jnp.float32),
        mesh=mesh,
        scratch_types=[
            pltpu.VMEM((b_per_w * CTX,), jnp.int32),
            pltpu.VMEM((2 * rows_per_pass, _LANES), jnp.float32),
            pltpu.VMEM((b_per_w, D), jnp.float32),
            pltpu.SemaphoreType.DMA,
        ],
    )
    def sc_pool(idx_hbm, emb_hbm, out_hbm, idx_v, rows_v, acc_v, sem):
        wid = lax.axis_index("s") * NC + lax.axis_index("c")
        pltpu.sync_copy(
            idx_hbm.at[pl.ds(wid * b_per_w * CTX, b_per_w * CTX)], idx_v
        )

        def fire(p, slot):
            handles = []
            pos = 0
            while pos < rows_per_pass:
                n = min(_LANES, rows_per_pass - pos)
                handles.append(
                    pltpu.async_copy(
                        emb_hbm.at[idx_v.at[pl.ds(p * rows_per_pass + pos, n)]],
                        rows_v.at[pl.ds(slot * rows_per_pass + pos, n)],
                        sem,
                    )
                )
                pos += n
            return handles

        pending = fire(0, 0)
        for p in range(passes):
            slot = p & 1
            for h in pending:
                h.wait()
            if p + 1 < passes:
                pending = fire(p + 1, slot ^ 1)
            for e in range(PASS_E):
                bi = p * PASS_E + e
                rbase = slot * rows_per_pass + e * CTX

                def body(r, accs, rbase=rbase):
                    return tuple(
                        a + rows_v[rbase + r, pl.ds(16 * c, 16)]
                        for c, a in enumerate(accs)
                    )

                z = jnp.zeros((16,), jnp.float32)
                accs = lax.fori_loop(0, CTX, body, (z,) * (D // 16))
                for c, a in enumerate(accs):
                    acc_v[bi, pl.ds(16 * c, 16)] = a
        pltpu.sync_copy(acc_v, out_hbm.at[pl.ds(wid * b_per_w, b_per_w)])

    return sc_pool


def _mm_kernel(s_ref, w_ref, b_ref, o_ref):
    lhs = jnp.concatenate([w_ref[...], b_ref[...]], axis=0)
    o_ref[...] = lax.dot_general(
        lhs,
        s_ref[...],
        dimension_numbers=(((0,), (1,)), ((), ())),
        preferred_element_type=jnp.float32,
    )


def _project(sums, W_t, b2d, ctx, tile_v):
    D, V = W_t.shape
    B = sums.shape[0]
    rhs = jnp.concatenate(
        [sums * (1.0 / ctx), jnp.ones((B, 1), jnp.float32)], axis=1
    )
    out_t = pl.pallas_call(
        _mm_kernel,
        grid=(pl.cdiv(V, tile_v),),
        in_specs=[
            pl.BlockSpec((B, D + 1), lambda j: (0, 0)),
            pl.BlockSpec((D, tile_v), lambda j: (0, j)),
            pl.BlockSpec((1, tile_v), lambda j: (0, j)),
        ],
        out_specs=pl.BlockSpec((tile_v, B), lambda j: (j, 0)),
        out_shape=jax.ShapeDtypeStruct((V, B), jnp.float32),
        compiler_params=pltpu.CompilerParams(
            dimension_semantics=("parallel",),
        ),
    )(rhs, W_t, b2d)
    return jnp.transpose(out_t)


def kernel(context_idxs, emb_table, W, b):
    B, CTX = context_idxs.shape
    V, D = emb_table.shape
    idx_lin = context_idxs.astype(jnp.int32).reshape(B * CTX)
    emb128 = _widen_table(emb_table.T, 16384)
    sums = _make_sc_pool(B, CTX, D)(idx_lin, emb128)
    return _project(sums, W.T, b.reshape(1, V), CTX, 2048)


# speedup vs baseline: 3.7624x; 1.0118x over previous
import functools

import jax
import jax.numpy as jnp
from jax import lax
from jax.experimental import pallas as pl
from jax.experimental.pallas import tpu as pltpu
from jax.experimental.pallas import tpu_sc as plsc

_LANES = 128


def _prep_kernel(et_ref, o_ref):
    blk = et_ref[...]
    o_ref[:, pl.ds(0, blk.shape[0])] = blk.T


def _widen_table(emb_t, tile_v):
    D, V = emb_t.shape
    return pl.pallas_call(
        _prep_kernel,
        grid=(pl.cdiv(V, tile_v),),
        in_specs=[pl.BlockSpec((D, tile_v), lambda j: (0, j))],
        out_specs=pl.BlockSpec((tile_v, _LANES), lambda j: (j, 0)),
        out_shape=jax.ShapeDtypeStruct((V, _LANES), jnp.float32),
        compiler_params=pltpu.CompilerParams(
            dimension_semantics=("parallel",),
        ),
    )(emb_t)


def _make_sc_pool(B, CTX, D, NC=2, NS=16, PASS_E=8):
    NW = NC * NS
    b_per_w = B // NW
    passes = b_per_w // PASS_E
    rows_per_pass = PASS_E * CTX
    mesh = plsc.VectorSubcoreMesh(core_axis_name="c", subcore_axis_name="s")

    @functools.partial(
        pl.kernel,
        out_type=jax.ShapeDtypeStruct((B, D), jnp.float32),
        mesh=mesh,
        scratch_types=[
            pltpu.VMEM((b_per_w * CTX,), jnp.int32),
            pltpu.VMEM((2 * rows_per_pass, _LANES), jnp.float32),
            pltpu.VMEM((b_per_w, D), jnp.float32),
            pltpu.SemaphoreType.DMA,
        ],
    )
    def sc_pool(idx_hbm, emb_hbm, out_hbm, idx_v, rows_v, acc_v, sem):
        wid = lax.axis_index("s") * NC + lax.axis_index("c")
        pltpu.sync_copy(
            idx_hbm.at[pl.ds(wid * b_per_w * CTX, b_per_w * CTX)], idx_v
        )

        def fire(p, slot):
            handles = []
            pos = 0
            while pos < rows_per_pass:
                n = min(_LANES, rows_per_pass - pos)
                handles.append(
                    pltpu.async_copy(
                        emb_hbm.at[idx_v.at[pl.ds(p * rows_per_pass + pos, n)]],
                        rows_v.at[pl.ds(slot * rows_per_pass + pos, n)],
                        sem,
                    )
                )
                pos += n
            return handles

        pending = fire(0, 0)
        for p in range(passes):
            slot = p & 1
            for h in pending:
                h.wait()
            if p + 1 < passes:
                pending = fire(p + 1, slot ^ 1)
            for e in range(PASS_E):
                bi = p * PASS_E + e
                rbase = slot * rows_per_pass + e * CTX

                def body(r, accs, rbase=rbase):
                    return tuple(
                        a + rows_v[rbase + r, pl.ds(16 * c, 16)]
                        for c, a in enumerate(accs)
                    )

                z = jnp.zeros((16,), jnp.float32)
                accs = lax.fori_loop(0, CTX, body, (z,) * (D // 16))
                for c, a in enumerate(accs):
                    acc_v[bi, pl.ds(16 * c, 16)] = a
        pltpu.sync_copy(acc_v, out_hbm.at[pl.ds(wid * b_per_w, b_per_w)])

    return sc_pool


def _mm_kernel(s_ref, w_ref, b_ref, o_ref):
    lhs = jnp.concatenate([w_ref[...], b_ref[...]], axis=0)
    o_ref[...] = lax.dot_general(
        lhs,
        s_ref[...],
        dimension_numbers=(((0,), (1,)), ((), ())),
        preferred_element_type=jnp.float32,
    )


def _project(sums, W_t, b2d, ctx, tile_v):
    D, V = W_t.shape
    B = sums.shape[0]
    rhs = jnp.concatenate(
        [sums * (1.0 / ctx), jnp.ones((B, 1), jnp.float32)], axis=1
    )
    out_t = pl.pallas_call(
        _mm_kernel,
        grid=(pl.cdiv(V, tile_v),),
        in_specs=[
            pl.BlockSpec((B, D + 1), lambda j: (0, 0)),
            pl.BlockSpec((D, tile_v), lambda j: (0, j)),
            pl.BlockSpec((1, tile_v), lambda j: (0, j)),
        ],
        out_specs=pl.BlockSpec((tile_v, B), lambda j: (j, 0)),
        out_shape=jax.ShapeDtypeStruct((V, B), jnp.float32),
        compiler_params=pltpu.CompilerParams(
            dimension_semantics=("parallel",),
        ),
    )(rhs, W_t, b2d)
    return jnp.transpose(out_t)


def kernel(context_idxs, emb_table, W, b):
    B, CTX = context_idxs.shape
    V, D = emb_table.shape
    idx_lin = context_idxs.astype(jnp.int32).reshape(B * CTX)
    emb128 = _widen_table(emb_table.T, 16384)
    sums = _make_sc_pool(B, CTX, D)(idx_lin, emb128)
    return _project(sums, W.T, b.reshape(1, V), CTX, 4096)


# speedup vs baseline: 3.7828x; 1.0054x over previous
import functools

import jax
import jax.numpy as jnp
from jax import lax
from jax.experimental import pallas as pl
from jax.experimental.pallas import tpu as pltpu
from jax.experimental.pallas import tpu_sc as plsc

_LANES = 128


def _prep_kernel(et_ref, o_ref):
    blk = et_ref[...]
    o_ref[:, pl.ds(0, blk.shape[0])] = blk.T


def _widen_table(emb_t, tile_v):
    D, V = emb_t.shape
    return pl.pallas_call(
        _prep_kernel,
        grid=(pl.cdiv(V, tile_v),),
        in_specs=[pl.BlockSpec((D, tile_v), lambda j: (0, j))],
        out_specs=pl.BlockSpec((tile_v, _LANES), lambda j: (j, 0)),
        out_shape=jax.ShapeDtypeStruct((V, _LANES), jnp.float32),
        compiler_params=pltpu.CompilerParams(
            dimension_semantics=("parallel",),
        ),
    )(emb_t)


def _make_sc_pool(B, CTX, D, NC=2, NS=16, PASS_E=8):
    NW = NC * NS
    b_per_w = B // NW
    passes = b_per_w // PASS_E
    rows_per_pass = PASS_E * CTX
    mesh = plsc.VectorSubcoreMesh(core_axis_name="c", subcore_axis_name="s")

    @functools.partial(
        pl.kernel,
        out_type=jax.ShapeDtypeStruct((B, D), jnp.float32),
        mesh=mesh,
        scratch_types=[
            pltpu.VMEM((b_per_w * CTX,), jnp.int32),
            pltpu.VMEM((2 * rows_per_pass, _LANES), jnp.float32),
            pltpu.VMEM((b_per_w, D), jnp.float32),
            pltpu.SemaphoreType.DMA,
        ],
    )
    def sc_pool(idx_hbm, emb_hbm, out_hbm, idx_v, rows_v, acc_v, sem):
        wid = lax.axis_index("s") * NC + lax.axis_index("c")
        pltpu.sync_copy(
            idx_hbm.at[pl.ds(wid * b_per_w * CTX, b_per_w * CTX)], idx_v
        )

        def fire(p, slot):
            handles = []
            pos = 0
            while pos < rows_per_pass:
                n = min(_LANES, rows_per_pass - pos)
                handles.append(
                    pltpu.async_copy(
                        emb_hbm.at[idx_v.at[pl.ds(p * rows_per_pass + pos, n)]],
                        rows_v.at[pl.ds(slot * rows_per_pass + pos, n)],
                        sem,
                    )
                )
                pos += n
            return handles

        pending = fire(0, 0)
        for p in range(passes):
            slot = p & 1
            for h in pending:
                h.wait()
            if p + 1 < passes:
                pending = fire(p + 1, slot ^ 1)
            for e in range(PASS_E):
                bi = p * PASS_E + e
                rbase = slot * rows_per_pass + e * CTX

                def body(r, accs, rbase=rbase):
                    return tuple(
                        a + rows_v[rbase + r, pl.ds(16 * c, 16)]
                        for c, a in enumerate(accs)
                    )

                z = jnp.zeros((16,), jnp.float32)
                accs = lax.fori_loop(0, CTX, body, (z,) * (D // 16))
                for c, a in enumerate(accs):
                    acc_v[bi, pl.ds(16 * c, 16)] = a
        pltpu.sync_copy(acc_v, out_hbm.at[pl.ds(wid * b_per_w, b_per_w)])

    return sc_pool


def _mm_kernel(s_ref, w_ref, b_ref, o_ref):
    lhs = jnp.concatenate([w_ref[...], b_ref[...]], axis=0)
    o_ref[...] = lax.dot_general(
        lhs,
        s_ref[...],
        dimension_numbers=(((0,), (1,)), ((), ())),
        preferred_element_type=jnp.float32,
    )


def _project(sums, W_t, b2d, ctx, tile_v):
    D, V = W_t.shape
    B = sums.shape[0]
    rhs = jnp.concatenate(
        [sums * (1.0 / ctx), jnp.ones((B, 1), jnp.float32)], axis=1
    )
    out_t = pl.pallas_call(
        _mm_kernel,
        grid=(pl.cdiv(V, tile_v),),
        in_specs=[
            pl.BlockSpec((B, D + 1), lambda j: (0, 0)),
            pl.BlockSpec((D, tile_v), lambda j: (0, j)),
            pl.BlockSpec((1, tile_v), lambda j: (0, j)),
        ],
        out_specs=pl.BlockSpec((tile_v, B), lambda j: (j, 0)),
        out_shape=jax.ShapeDtypeStruct((V, B), jnp.float32),
        compiler_params=pltpu.CompilerParams(
            dimension_semantics=("parallel",),
        ),
    )(rhs, W_t, b2d)
    return jnp.transpose(out_t)


def kernel(context_idxs, emb_table, W, b):
    B, CTX = context_idxs.shape
    V, D = emb_table.shape
    idx_lin = context_idxs.astype(jnp.int32).reshape(B * CTX)
    emb128 = _widen_table(emb_table.T, 32768)
    sums = _make_sc_pool(B, CTX, D)(idx_lin, emb128)
    return _project(sums, W.T, b.reshape(1, V), CTX, 4096)
